# Initial kernel scaffold; baseline (speedup 1.0000x reference)
import functools
import math

import jax
import jax.numpy as jnp
from jax import lax
from jax.experimental import pallas as pl
from jax.experimental.pallas import tpu as pltpu
from jax.experimental.pallas import tpu_sc as plsc

_B = 2048
_C = 4
_M = 4096
_P = 16384

_NC = 2
_NS = 16
_NW = _NC * _NS
_CH = _P // _NW
_LANES = 16

_LOG2PI = math.log(2.0 * math.pi)


def _sc_gather_body(edges_hbm, mu_hbm, ls_hbm, gmu_hbm, gls_hbm,
                    mu_v, ls_v, edges_v, gmu_v, gls_v):
    wid = lax.axis_index("s") * _NC + lax.axis_index("c")
    pltpu.sync_copy(mu_hbm, mu_v)
    pltpu.sync_copy(ls_hbm, ls_v)
    for c in range(_C):
        pltpu.sync_copy(edges_hbm.at[c, wid], edges_v.at[c])
    for c in range(_C):
        for i in range(_CH // _LANES):
            sl = pl.ds(i * _LANES, _LANES)
            idx = edges_v[c, sl] + c * _M
            gmu_v[c, sl] = plsc.load_gather(mu_v, [idx])
            gls_v[c, sl] = plsc.load_gather(ls_v, [idx])
    for c in range(_C):
        pltpu.sync_copy(gmu_v.at[c], gmu_hbm.at[c, wid])
        pltpu.sync_copy(gls_v.at[c], gls_hbm.at[c, wid])


_sc_gather = functools.partial(
    pl.kernel,
    out_type=[
        jax.ShapeDtypeStruct((_C, _NW, _CH), jnp.float32),
        jax.ShapeDtypeStruct((_C, _NW, _CH), jnp.float32),
    ],
    mesh=plsc.VectorSubcoreMesh(core_axis_name="c", subcore_axis_name="s"),
    scratch_types=[
        pltpu.VMEM((_C * _M,), jnp.float32),
        pltpu.VMEM((_C * _M,), jnp.float32),
        pltpu.VMEM((_C, _CH), jnp.int32),
        pltpu.VMEM((_C, _CH), jnp.float32),
        pltpu.VMEM((_C, _CH), jnp.float32),
    ],
)(_sc_gather_body)


_BB = 512
_PB = 2048


def _tc_combine_body(x_ref, gmu_ref, gls_ref, out_ref):
    xb = x_ref[...]
    gmu = gmu_ref[...]
    gls = gls_ref[...]
    inv2 = jnp.exp(-2.0 * gls)
    a2 = -0.5 * inv2
    b2 = gmu * inv2
    drow = (jnp.sum(-0.5 * gmu * gmu * inv2 - gls, axis=0, keepdims=True)
            - _C * 0.5 * _LOG2PI)
    acc = jnp.dot(xb * xb, a2, preferred_element_type=jnp.float32,
                  precision=lax.Precision.HIGHEST)
    acc = acc + jnp.dot(xb, b2, preferred_element_type=jnp.float32,
                        precision=lax.Precision.HIGHEST)
    out_ref[...] = acc + drow


_tc_combine = pl.pallas_call(
    _tc_combine_body,
    grid=(_B // _BB, _P // _PB),
    in_specs=[
        pl.BlockSpec((_BB, _C), lambda i, j: (i, 0)),
        pl.BlockSpec((_C, _PB), lambda i, j: (0, j)),
        pl.BlockSpec((_C, _PB), lambda i, j: (0, j)),
    ],
    out_specs=pl.BlockSpec((_BB, _PB), lambda i, j: (i, j)),
    out_shape=jax.ShapeDtypeStruct((_B, _P), jnp.float32),
)


def kernel(x, edges, mu, log_sigma):
    edges3 = edges.reshape(_C, _NW, _CH)
    gmu3, gls3 = _sc_gather(edges3, mu.reshape(-1), log_sigma.reshape(-1))
    return _tc_combine(x, gmu3.reshape(_C, _P), gls3.reshape(_C, _P))


# speedup vs baseline: 11.9212x; 11.9212x over previous
import functools
import math

import jax
import jax.numpy as jnp
from jax import lax
from jax.experimental import pallas as pl
from jax.experimental.pallas import tpu as pltpu
from jax.experimental.pallas import tpu_sc as plsc

_B = 2048
_C = 4
_M = 4096
_P = 16384

_NC = 2
_NS = 16
_NW = _NC * _NS
_CH = _P // _NW
_LANES = 16

_LOG2PI = math.log(2.0 * math.pi)


def _sc_gather_body(edges_hbm, mu_hbm, ls_hbm, gmu_hbm, gls_hbm,
                    mu_v, ls_v, edges_v, gmu_v, gls_v):
    wid = lax.axis_index("s") * _NC + lax.axis_index("c")
    pltpu.sync_copy(mu_hbm, mu_v)
    pltpu.sync_copy(ls_hbm, ls_v)
    for c in range(_C):
        pltpu.sync_copy(edges_hbm.at[c, wid], edges_v.at[c])
    for c in range(_C):
        for i in range(_CH // _LANES):
            sl = pl.ds(i * _LANES, _LANES)
            idx = edges_v[c, sl] + c * _M
            gmu_v[c, sl] = plsc.load_gather(mu_v, [idx])
            gls_v[c, sl] = plsc.load_gather(ls_v, [idx])
    for c in range(_C):
        pltpu.sync_copy(gmu_v.at[c], gmu_hbm.at[c, wid])
        pltpu.sync_copy(gls_v.at[c], gls_hbm.at[c, wid])


@functools.lru_cache(maxsize=1)
def _sc_gather():
    return pl.kernel(
        _sc_gather_body,
        out_type=[
            jax.ShapeDtypeStruct((_C, _NW, _CH), jnp.float32),
            jax.ShapeDtypeStruct((_C, _NW, _CH), jnp.float32),
        ],
        mesh=plsc.VectorSubcoreMesh(core_axis_name="c", subcore_axis_name="s"),
        compiler_params=pltpu.CompilerParams(needs_layout_passes=False),
        scratch_types=[
            pltpu.VMEM((_C * _M,), jnp.float32),
            pltpu.VMEM((_C * _M,), jnp.float32),
            pltpu.VMEM((_C, _CH), jnp.int32),
            pltpu.VMEM((_C, _CH), jnp.float32),
            pltpu.VMEM((_C, _CH), jnp.float32),
        ],
    )


_BB = 512
_PB = 2048


def _tc_combine_body(x_ref, gmu_ref, gls_ref, out_ref):
    xb = x_ref[...]
    gmu = gmu_ref[...]
    gls = gls_ref[...]
    inv2 = jnp.exp(-2.0 * gls)
    a2 = -0.5 * inv2
    b2 = gmu * inv2
    drow = (jnp.sum(-0.5 * gmu * gmu * inv2 - gls, axis=0, keepdims=True)
            - _C * 0.5 * _LOG2PI)
    acc = jnp.dot(xb * xb, a2, preferred_element_type=jnp.float32,
                  precision=lax.Precision.HIGHEST)
    acc = acc + jnp.dot(xb, b2, preferred_element_type=jnp.float32,
                        precision=lax.Precision.HIGHEST)
    out_ref[...] = acc + drow


_tc_combine = pl.pallas_call(
    _tc_combine_body,
    grid=(_B // _BB, _P // _PB),
    in_specs=[
        pl.BlockSpec((_BB, _C), lambda i, j: (i, 0)),
        pl.BlockSpec((_C, _PB), lambda i, j: (0, j)),
        pl.BlockSpec((_C, _PB), lambda i, j: (0, j)),
    ],
    out_specs=pl.BlockSpec((_BB, _PB), lambda i, j: (i, j)),
    out_shape=jax.ShapeDtypeStruct((_B, _P), jnp.float32),
)


def kernel(x, edges, mu, log_sigma):
    edges3 = edges.reshape(_C, _NW, _CH)
    gmu3, gls3 = _sc_gather()(edges3, mu.reshape(-1), log_sigma.reshape(-1))
    return _tc_combine(x, gmu3.reshape(_C, _P), gls3.reshape(_C, _P))


# speedup vs baseline: 35.6217x; 2.9881x over previous
import functools
import math

import jax
import jax.numpy as jnp
from jax import lax
from jax.experimental import pallas as pl
from jax.experimental.pallas import tpu as pltpu
from jax.experimental.pallas import tpu_sc as plsc

_B = 2048
_C = 4
_M = 4096
_P = 16384

_NC = 2
_NS = 16
_NW = _NC * _NS
_CH = _P // _NW
_LANES = 16

_LOG2PI = math.log(2.0 * math.pi)


def _sc_gather_body(edges_hbm, mu_hbm, ls_hbm, gmu_hbm, gls_hbm,
                    mu_v, ls_v, edges_v, gmu_v, gls_v):
    wid = lax.axis_index("s") * _NC + lax.axis_index("c")
    pltpu.sync_copy(mu_hbm, mu_v)
    pltpu.sync_copy(ls_hbm, ls_v)
    for c in range(_C):
        pltpu.sync_copy(edges_hbm.at[c, wid], edges_v.at[c])
    for c in range(_C):
        for i in range(_CH // _LANES):
            sl = pl.ds(i * _LANES, _LANES)
            idx = edges_v[c, sl] + c * _M
            gmu_v[c, sl] = plsc.load_gather(mu_v, [idx])
            gls_v[c, sl] = plsc.load_gather(ls_v, [idx])
    for c in range(_C):
        pltpu.sync_copy(gmu_v.at[c], gmu_hbm.at[c, wid])
        pltpu.sync_copy(gls_v.at[c], gls_hbm.at[c, wid])


@functools.lru_cache(maxsize=1)
def _sc_gather():
    return pl.kernel(
        _sc_gather_body,
        out_type=[
            jax.ShapeDtypeStruct((_C, _NW, _CH), jnp.float32),
            jax.ShapeDtypeStruct((_C, _NW, _CH), jnp.float32),
        ],
        mesh=plsc.VectorSubcoreMesh(core_axis_name="c", subcore_axis_name="s"),
        compiler_params=pltpu.CompilerParams(needs_layout_passes=False),
        scratch_types=[
            pltpu.VMEM((_C * _M,), jnp.float32),
            pltpu.VMEM((_C * _M,), jnp.float32),
            pltpu.VMEM((_C, _CH), jnp.int32),
            pltpu.VMEM((_C, _CH), jnp.float32),
            pltpu.VMEM((_C, _CH), jnp.float32),
        ],
    )


_BB = 512
_PB = 2048


def _tc_combine_body(xcat_ref, gmu_ref, gls_ref, out_ref):
    xcat = xcat_ref[...]
    gmu = gmu_ref[...]
    gls = gls_ref[...]
    inv2 = jnp.exp(-2.0 * gls)
    a2 = -0.5 * inv2
    b2 = gmu * inv2
    coef = jnp.concatenate([a2, b2], axis=0)
    drow = (jnp.sum(-0.5 * gmu * gmu * inv2 - gls, axis=0, keepdims=True)
            - _C * 0.5 * _LOG2PI)
    acc = jnp.dot(xcat, coef, preferred_element_type=jnp.float32)
    out_ref[...] = acc + drow


_tc_combine = pl.pallas_call(
    _tc_combine_body,
    grid=(_B // _BB, _P // _PB),
    in_specs=[
        pl.BlockSpec((_BB, 2 * _C), lambda i, j: (i, 0)),
        pl.BlockSpec((_C, _PB), lambda i, j: (0, j)),
        pl.BlockSpec((_C, _PB), lambda i, j: (0, j)),
    ],
    out_specs=pl.BlockSpec((_BB, _PB), lambda i, j: (i, j)),
    out_shape=jax.ShapeDtypeStruct((_B, _P), jnp.float32),
)


def kernel(x, edges, mu, log_sigma):
    edges3 = edges.reshape(_C, _NW, _CH)
    gmu3, gls3 = _sc_gather()(edges3, mu.reshape(-1), log_sigma.reshape(-1))
    xcat = jnp.concatenate([x * x, x], axis=1)
    return _tc_combine(xcat, gmu3.reshape(_C, _P), gls3.reshape(_C, _P))
